# Initial kernel scaffold; baseline (speedup 1.0000x reference)
import jax
import jax.numpy as jnp
from jax.experimental import pallas as pl


def kernel(inputs_ids, token_type_ids, word_table, token_type_table, position_table):
    raise NotImplementedError("write your pallas kernel here")


# speedup vs baseline: 3.1511x; 3.1511x over previous
import functools

import jax
import jax.numpy as jnp
from jax import lax
from jax.experimental import pallas as pl
from jax.experimental.pallas import tpu as pltpu
from jax.experimental.pallas import tpu_sc as plsc

VOCAB = 100000
HIDDEN = 128
S = 200
B = 1024

NC = 2
NS = 16
NW = NC * NS
N_TOK = B * S
TOK_PER_W = N_TOK // NW
CHUNK = 128
N_CHUNKS = TOK_PER_W // CHUNK
NSL = HIDDEN // 16


def _body(ids_hbm, tt_hbm, word_hbm, ttt_hbm, pos_hbm, out_hbm,
          comb_v, ttrow_v, idx_v, tts_v, rows_v, sem):
    wid = lax.axis_index("s") * NC + lax.axis_index("c")

    pltpu.sync_copy(pos_hbm.at[pl.ds(0, S)], comb_v.at[pl.ds(0, S)])
    pltpu.sync_copy(pos_hbm.at[pl.ds(0, S)], comb_v.at[pl.ds(S, S)])
    pltpu.sync_copy(ttt_hbm, ttrow_v)

    def build(s, _):
        for j in range(NSL):
            sl = pl.ds(16 * j, 16)
            comb_v[s, sl] += ttrow_v[0, sl]
            comb_v[S + s, sl] += ttrow_v[1, sl]
        return 0

    lax.fori_loop(0, S, build, 0)

    def chunk_step(c, _):
        base = wid * TOK_PER_W + c * CHUNK
        pltpu.sync_copy(ids_hbm.at[pl.ds(base, CHUNK)], idx_v)
        pltpu.sync_copy(tt_hbm.at[pl.ds(base, CHUNK)], tts_v)
        pltpu.async_copy(word_hbm.at[idx_v], rows_v, sem).wait()

        def tok_group(g, _):
            g16 = g * 16
            tv = tts_v[pl.ds(g16, 16)]
            for i in range(16):
                s = g16 + i
                p = lax.rem(base + s, S)
                k = tv[i] * S + p
                for j in range(NSL):
                    sl = pl.ds(16 * j, 16)
                    rows_v[s, sl] += comb_v[k, sl]
            return 0

        lax.fori_loop(0, CHUNK // 16, tok_group, 0)
        pltpu.sync_copy(rows_v, out_hbm.at[pl.ds(base, CHUNK)])
        return 0

    lax.fori_loop(0, N_CHUNKS, chunk_step, 0)


@jax.jit
def _embed(ids_flat, tt_flat, word_table, token_type_table, position_table):
    mesh = plsc.VectorSubcoreMesh(core_axis_name="c", subcore_axis_name="s")
    k = functools.partial(
        pl.kernel,
        mesh=mesh,
        out_type=jax.ShapeDtypeStruct((N_TOK, HIDDEN), jnp.float32),
        scratch_types=[
            pltpu.VMEM((2 * S, HIDDEN), jnp.float32),
            pltpu.VMEM((2, HIDDEN), jnp.float32),
            pltpu.VMEM((CHUNK,), jnp.int32),
            pltpu.VMEM((CHUNK,), jnp.int32),
            pltpu.VMEM((CHUNK, HIDDEN), jnp.float32),
            pltpu.SemaphoreType.DMA,
        ],
    )(_body)
    return k(ids_flat, tt_flat, word_table, token_type_table, position_table)


def kernel(inputs_ids, token_type_ids, word_table, token_type_table, position_table):
    ids_flat = inputs_ids.reshape(-1).astype(jnp.int32)
    tt_flat = token_type_ids.reshape(-1).astype(jnp.int32)
    out = _embed(ids_flat, tt_flat, word_table, token_type_table, position_table)
    return (out.reshape(inputs_ids.shape[0], inputs_ids.shape[1], HIDDEN), word_table)


# speedup vs baseline: 4.4759x; 1.4204x over previous
import functools

import jax
import jax.numpy as jnp
from jax import lax
from jax.experimental import pallas as pl
from jax.experimental.pallas import tpu as pltpu
from jax.experimental.pallas import tpu_sc as plsc

VOCAB = 100000
HIDDEN = 128
S = 200
B = 1024

NC = 2
NS = 16
NW = NC * NS
N_TOK = B * S
TOK_PER_W = N_TOK // NW
CHUNK = 80
N_CHUNKS = TOK_PER_W // CHUNK
NBUF = 4
N_OUTER = N_CHUNKS // NBUF
NSL = HIDDEN // 16
ROW_BYTES = HIDDEN * 4


def _body(ids_hbm, tt_hbm, word_hbm, ttt_hbm, pos_hbm, out_hbm,
          comb_v, ttrow_v, idx_v, tts_v, rows_v,
          gs0, gs1, gs2, gs3, ws0, ws1, ws2, ws3):
    gsem = (gs0, gs1, gs2, gs3)
    wsem = (ws0, ws1, ws2, ws3)
    wid = lax.axis_index("s") * NC + lax.axis_index("c")
    wbase = wid * TOK_PER_W

    pltpu.sync_copy(ids_hbm.at[pl.ds(wbase, TOK_PER_W)], idx_v)
    pltpu.sync_copy(tt_hbm.at[pl.ds(wbase, TOK_PER_W)], tts_v)

    pltpu.sync_copy(pos_hbm.at[pl.ds(0, S)], comb_v.at[pl.ds(0, S)])
    pltpu.sync_copy(pos_hbm.at[pl.ds(0, S)], comb_v.at[pl.ds(S, S)])
    pltpu.sync_copy(ttt_hbm, ttrow_v)

    def build(s, _):
        for j in range(NSL):
            sl = pl.ds(16 * j, 16)
            comb_v[s, sl] += ttrow_v[0, sl]
            comb_v[S + s, sl] += ttrow_v[1, sl]
        return 0

    lax.fori_loop(0, S, build, 0)

    def start_gather(c, b):
        pltpu.make_async_copy(
            word_hbm.at[idx_v.at[pl.ds(c * CHUNK, CHUNK)]],
            rows_v.at[b], gsem[b]).start()

    def wait_gather(b):
        pltpu.make_async_copy(
            word_hbm.at[idx_v.at[pl.ds(0, CHUNK)]],
            rows_v.at[b], gsem[b]).wait()

    def start_write(c, b):
        pltpu.make_async_copy(
            rows_v.at[b], out_hbm.at[pl.ds(wbase + c * CHUNK, CHUNK)],
            wsem[b]).start()

    def wait_write(b):
        pltpu.make_async_copy(
            rows_v.at[b], out_hbm.at[pl.ds(wbase, CHUNK)], wsem[b]).wait()

    start_gather(0, 0)
    start_gather(1, 1)

    def outer(i, _):
        for b in range(NBUF):
            c = i * NBUF + b
            wait_gather(b)

            def tok_group(g, _):
                g16 = g * 16
                tv = tts_v[pl.ds(c * CHUNK + g16, 16)]
                for t in range(16):
                    s = g16 + t
                    p = lax.rem(c * CHUNK + s, S)
                    k = tv[t] * S + p
                    for j in range(NSL):
                        sl = pl.ds(16 * j, 16)
                        rows_v[b, s, sl] += comb_v[k, sl]
                return 0

            lax.fori_loop(0, CHUNK // 16, tok_group, 0)
            start_write(c, b)

            cn = c + 2
            bn = (b + 2) % NBUF

            @pl.when(cn < N_CHUNKS)
            def _():
                @pl.when(cn >= NBUF)
                def _():
                    wait_write(bn)
                start_gather(cn, bn)
        return 0

    lax.fori_loop(0, N_OUTER, outer, 0)
    for b in range(NBUF):
        wait_write(b)


@jax.jit
def _embed(ids_flat, tt_flat, word_table, token_type_table, position_table):
    mesh = plsc.VectorSubcoreMesh(core_axis_name="c", subcore_axis_name="s")
    k = functools.partial(
        pl.kernel,
        mesh=mesh,
        out_type=jax.ShapeDtypeStruct((N_TOK, HIDDEN), jnp.float32),
        scratch_types=[
            pltpu.VMEM((2 * S, HIDDEN), jnp.float32),
            pltpu.VMEM((2, HIDDEN), jnp.float32),
            pltpu.VMEM((TOK_PER_W,), jnp.int32),
            pltpu.VMEM((TOK_PER_W,), jnp.int32),
            pltpu.VMEM((NBUF, CHUNK, HIDDEN), jnp.float32),
        ] + [pltpu.SemaphoreType.DMA] * (2 * NBUF),
    )(_body)
    return k(ids_flat, tt_flat, word_table, token_type_table, position_table)


def kernel(inputs_ids, token_type_ids, word_table, token_type_table, position_table):
    ids_flat = inputs_ids.reshape(-1).astype(jnp.int32)
    tt_flat = token_type_ids.reshape(-1).astype(jnp.int32)
    out = _embed(ids_flat, tt_flat, word_table, token_type_table, position_table)
    return (out.reshape(inputs_ids.shape[0], inputs_ids.shape[1], HIDDEN), word_table)


# speedup vs baseline: 7.6722x; 1.7141x over previous
import functools

import jax
import jax.numpy as jnp
from jax import lax
from jax.experimental import pallas as pl
from jax.experimental.pallas import tpu as pltpu
from jax.experimental.pallas import tpu_sc as plsc

VOCAB = 100000
HIDDEN = 128
S = 200
B = 1024

NC = 2
NS = 16
NW = NC * NS
N_TOK = B * S
TOK_PER_W = N_TOK // NW
CHUNK = 80
N_CHUNKS = TOK_PER_W // CHUNK
NBUF = 4
N_OUTER = N_CHUNKS // NBUF
NSL = HIDDEN // 16
ROW_BYTES = HIDDEN * 4


def _body(ids_hbm, tt_hbm, word_hbm, ttt_hbm, pos_hbm, out_hbm,
          comb_v, ttrow_v, idx_v, tts_v, rows_v,
          gs0, gs1, gs2, gs3, ws0, ws1, ws2, ws3):
    gsem = (gs0, gs1, gs2, gs3)
    wsem = (ws0, ws1, ws2, ws3)
    wid = lax.axis_index("s") * NC + lax.axis_index("c")
    wbase = wid * TOK_PER_W

    pltpu.sync_copy(ids_hbm.at[pl.ds(wbase, TOK_PER_W)], idx_v)
    pltpu.sync_copy(tt_hbm.at[pl.ds(wbase, TOK_PER_W)], tts_v)

    pltpu.sync_copy(pos_hbm.at[pl.ds(0, S)], comb_v.at[pl.ds(0, S)])
    pltpu.sync_copy(pos_hbm.at[pl.ds(0, S)], comb_v.at[pl.ds(S, S)])
    pltpu.sync_copy(ttt_hbm, ttrow_v)

    def build(s, _):
        for j in range(NSL):
            sl = pl.ds(16 * j, 16)
            comb_v[s, sl] += ttrow_v[0, sl]
            comb_v[S + s, sl] += ttrow_v[1, sl]
        return 0

    lax.fori_loop(0, S, build, 0)

    def start_gather(c, b):
        pltpu.make_async_copy(
            word_hbm.at[idx_v.at[pl.ds(c * CHUNK, CHUNK)]],
            rows_v.at[b], gsem[b]).start()

    def wait_gather(b):
        pltpu.make_async_copy(
            word_hbm.at[idx_v.at[pl.ds(0, CHUNK)]],
            rows_v.at[b], gsem[b]).wait()

    def start_write(c, b):
        pltpu.make_async_copy(
            rows_v.at[b], out_hbm.at[pl.ds(wbase + c * CHUNK, CHUNK)],
            wsem[b]).start()

    def wait_write(b):
        pltpu.make_async_copy(
            rows_v.at[b], out_hbm.at[pl.ds(wbase, CHUNK)], wsem[b]).wait()

    start_gather(0, 0)
    start_gather(1, 1)

    def outer(i, _):
        for b in range(NBUF):
            c = i * NBUF + b
            wait_gather(b)

            def tok_group(g, _):
                g16 = g * 16
                tv = tts_v[pl.ds(c * CHUNK + g16, 16)]
                for t in range(16):
                    s = g16 + t
                    p = lax.rem(c * CHUNK + s, S)
                    k = tv[t] * S + p
                    cs = [comb_v[k, pl.ds(16 * j, 16)] for j in range(NSL)]
                    rs = [rows_v[b, s, pl.ds(16 * j, 16)] for j in range(NSL)]
                    for j in range(NSL):
                        rows_v[b, s, pl.ds(16 * j, 16)] = rs[j] + cs[j]
                return 0

            lax.fori_loop(0, CHUNK // 16, tok_group, 0)
            start_write(c, b)

            cn = c + 2
            bn = (b + 2) % NBUF

            @pl.when(cn < N_CHUNKS)
            def _():
                @pl.when(cn >= NBUF)
                def _():
                    wait_write(bn)
                start_gather(cn, bn)
        return 0

    lax.fori_loop(0, N_OUTER, outer, 0)
    for b in range(NBUF):
        wait_write(b)


@jax.jit
def _embed(ids_flat, tt_flat, word_table, token_type_table, position_table):
    mesh = plsc.VectorSubcoreMesh(core_axis_name="c", subcore_axis_name="s")
    k = functools.partial(
        pl.kernel,
        mesh=mesh,
        out_type=jax.ShapeDtypeStruct((N_TOK, HIDDEN), jnp.float32),
        scratch_types=[
            pltpu.VMEM((2 * S, HIDDEN), jnp.float32),
            pltpu.VMEM((2, HIDDEN), jnp.float32),
            pltpu.VMEM((TOK_PER_W,), jnp.int32),
            pltpu.VMEM((TOK_PER_W,), jnp.int32),
            pltpu.VMEM((NBUF, CHUNK, HIDDEN), jnp.float32),
        ] + [pltpu.SemaphoreType.DMA] * (2 * NBUF),
    )(_body)
    return k(ids_flat, tt_flat, word_table, token_type_table, position_table)


def kernel(inputs_ids, token_type_ids, word_table, token_type_table, position_table):
    ids_flat = inputs_ids.reshape(-1).astype(jnp.int32)
    tt_flat = token_type_ids.reshape(-1).astype(jnp.int32)
    out = _embed(ids_flat, tt_flat, word_table, token_type_table, position_table)
    return (out.reshape(inputs_ids.shape[0], inputs_ids.shape[1], HIDDEN), word_table)
